# baseline (speedup 1.0000x reference)
import functools

import jax
import jax.numpy as jnp
from jax import lax
from jax.experimental import pallas as pl
from jax.experimental.pallas import tpu as pltpu
from jax.experimental.pallas import tpu_sc as plsc

B, N, D_MODEL = 16, 1024, 128
ROWS = B * N
TC_BLOCK_ROWS = 512
TC_NBLK = ROWS // TC_BLOCK_ROWS

NUM_WORKERS = 32
ROWS_PER_WORKER = ROWS // NUM_WORKERS


def _count_kernel(d_ref, idx_ref):
    d = d_ref[0]
    hit = jnp.logical_or(d == 1, d == -1)
    idx_ref[0, 0, :] = jnp.sum(hit.astype(jnp.int32), axis=-1)


def _centrality_indices(distances):
    d3 = distances.reshape(TC_NBLK, TC_BLOCK_ROWS, N)
    idx = pl.pallas_call(
        _count_kernel,
        grid=(TC_NBLK,),
        in_specs=[pl.BlockSpec((1, TC_BLOCK_ROWS, N), lambda i: (i, 0, 0))],
        out_specs=pl.BlockSpec((1, 1, TC_BLOCK_ROWS), lambda i: (i, 0, 0)),
        out_shape=jax.ShapeDtypeStruct((TC_NBLK, 1, TC_BLOCK_ROWS), jnp.int32),
    )(d3)
    return idx.reshape(ROWS)


def _sc_gather(table, idx):
    mesh = plsc.VectorSubcoreMesh(core_axis_name="c", subcore_axis_name="s")

    @functools.partial(
        pl.kernel,
        mesh=mesh,
        out_type=jax.ShapeDtypeStruct((ROWS, D_MODEL), jnp.float32),
        scratch_types=[
            pltpu.VMEM((ROWS_PER_WORKER,), jnp.int32),
            pltpu.VMEM((ROWS_PER_WORKER, D_MODEL), jnp.float32),
            pltpu.SemaphoreType.DMA,
        ],
    )
    def gather_k(table_hbm, idx_hbm, out_hbm, idx_v, rows_v, sem):
        wid = lax.axis_index("s") * 2 + lax.axis_index("c")
        base = wid * ROWS_PER_WORKER
        pltpu.sync_copy(idx_hbm.at[pl.ds(base, ROWS_PER_WORKER)], idx_v)
        pltpu.async_copy(table_hbm.at[idx_v], rows_v, sem).wait()
        pltpu.sync_copy(rows_v, out_hbm.at[pl.ds(base, ROWS_PER_WORKER)])

    return gather_k(table, idx)


def kernel(distances, table):
    idx = _centrality_indices(distances)
    out = _sc_gather(table, idx)
    return out.reshape(B, N, D_MODEL)


# speedup vs baseline: 1.0298x; 1.0298x over previous
import functools

import jax
import jax.numpy as jnp
from jax import lax
from jax.experimental import pallas as pl
from jax.experimental.pallas import tpu as pltpu
from jax.experimental.pallas import tpu_sc as plsc

B, N, D_MODEL = 16, 1024, 128
ROWS = B * N
TC_BLOCK_ROWS = 1024
TC_NBLK = ROWS // TC_BLOCK_ROWS

NUM_WORKERS = 32
ROWS_PER_WORKER = ROWS // NUM_WORKERS
GATHER_CHUNK = 128
NCHUNK = ROWS_PER_WORKER // GATHER_CHUNK


def _count_kernel(d_ref, idx_ref):
    d = d_ref[0]
    hit = jnp.logical_or(d == 1, d == -1)
    idx_ref[0, 0, :] = jnp.sum(hit.astype(jnp.int32), axis=-1)


def _centrality_indices(distances):
    d3 = distances.reshape(TC_NBLK, TC_BLOCK_ROWS, N)
    idx = pl.pallas_call(
        _count_kernel,
        grid=(TC_NBLK,),
        in_specs=[pl.BlockSpec((1, TC_BLOCK_ROWS, N), lambda i: (i, 0, 0))],
        out_specs=pl.BlockSpec((1, 1, TC_BLOCK_ROWS), lambda i: (i, 0, 0)),
        out_shape=jax.ShapeDtypeStruct((TC_NBLK, 1, TC_BLOCK_ROWS), jnp.int32),
    )(d3)
    return idx.reshape(ROWS)


def _sc_gather(table, idx):
    mesh = plsc.VectorSubcoreMesh(core_axis_name="c", subcore_axis_name="s")

    @functools.partial(
        pl.kernel,
        mesh=mesh,
        out_type=jax.ShapeDtypeStruct((ROWS, D_MODEL), jnp.float32),
        scratch_types=[
            pltpu.VMEM((ROWS_PER_WORKER,), jnp.int32),
            pltpu.VMEM((NCHUNK, GATHER_CHUNK, D_MODEL), jnp.float32),
            pltpu.SemaphoreType.DMA,
            pltpu.SemaphoreType.DMA,
        ],
    )
    def gather_k(table_hbm, idx_hbm, out_hbm, idx_v, rows_v, gsem, wsem):
        wid = lax.axis_index("s") * 2 + lax.axis_index("c")
        base = wid * ROWS_PER_WORKER
        pltpu.sync_copy(idx_hbm.at[pl.ds(base, ROWS_PER_WORKER)], idx_v)
        gathers = [
            pltpu.async_copy(
                table_hbm.at[idx_v.at[pl.ds(g * GATHER_CHUNK, GATHER_CHUNK)]],
                rows_v.at[g], gsem)
            for g in range(NCHUNK)
        ]
        writes = []
        for g in range(NCHUNK):
            gathers[g].wait()
            writes.append(pltpu.async_copy(
                rows_v.at[g],
                out_hbm.at[pl.ds(base + g * GATHER_CHUNK, GATHER_CHUNK)], wsem))
        for w in writes:
            w.wait()

    return gather_k(table, idx)


def kernel(distances, table):
    idx = _centrality_indices(distances)
    out = _sc_gather(table, idx)
    return out.reshape(B, N, D_MODEL)


# speedup vs baseline: 2.4485x; 2.3777x over previous
import functools

import jax
import jax.numpy as jnp
from jax import lax
from jax.experimental import pallas as pl
from jax.experimental.pallas import tpu as pltpu
from jax.experimental.pallas import tpu_sc as plsc

B, N, D_MODEL = 16, 1024, 128
MAX_DEGREE = 1025
ROWS = B * N
TC_BLOCK_ROWS = 1024
TC_NBLK = ROWS // TC_BLOCK_ROWS

NUM_WORKERS = 32
ROWS_PER_WORKER = ROWS // NUM_WORKERS
GATHER_CHUNK = 128
NCHUNK = ROWS_PER_WORKER // GATHER_CHUNK


def _count_kernel(d_ref, idx_ref):
    d = d_ref[0]
    hit = jnp.logical_or(d == 1, d == -1)
    idx_ref[0, 0, :] = jnp.sum(hit.astype(jnp.int32), axis=-1)


def _centrality_indices(distances):
    d3 = distances.reshape(TC_NBLK, TC_BLOCK_ROWS, N)
    idx = pl.pallas_call(
        _count_kernel,
        grid=(TC_NBLK,),
        in_specs=[pl.BlockSpec((1, TC_BLOCK_ROWS, N), lambda i: (i, 0, 0))],
        out_specs=pl.BlockSpec((1, 1, TC_BLOCK_ROWS), lambda i: (i, 0, 0)),
        out_shape=jax.ShapeDtypeStruct((TC_NBLK, 1, TC_BLOCK_ROWS), jnp.int32),
    )(d3)
    return idx.reshape(ROWS)


def _sc_gather(table, idx):
    mesh = plsc.VectorSubcoreMesh(core_axis_name="c", subcore_axis_name="s")

    @functools.partial(
        pl.kernel,
        mesh=mesh,
        out_type=jax.ShapeDtypeStruct((ROWS, D_MODEL), jnp.float32),
        scratch_types=[
            pltpu.VMEM((ROWS_PER_WORKER,), jnp.int32),
            pltpu.VMEM((NCHUNK, GATHER_CHUNK, D_MODEL), jnp.float32),
            pltpu.VMEM_SHARED((MAX_DEGREE, D_MODEL), jnp.float32),
            pltpu.SemaphoreType.DMA,
            pltpu.SemaphoreType.DMA,
        ],
    )
    def gather_k(table_hbm, idx_hbm, out_hbm, idx_v, rows_v, table_sp,
                 gsem, wsem):
        sid = lax.axis_index("s")
        wid = sid * 2 + lax.axis_index("c")
        base = wid * ROWS_PER_WORKER

        @pl.when(sid == 0)
        def _stage_table():
            pltpu.sync_copy(table_hbm, table_sp)

        pltpu.sync_copy(idx_hbm.at[pl.ds(base, ROWS_PER_WORKER)], idx_v)
        plsc.subcore_barrier()

        gathers = [
            pltpu.async_copy(
                table_sp.at[idx_v.at[pl.ds(g * GATHER_CHUNK, GATHER_CHUNK)]],
                rows_v.at[g], gsem)
            for g in range(NCHUNK)
        ]
        writes = []
        for g in range(NCHUNK):
            gathers[g].wait()
            writes.append(pltpu.async_copy(
                rows_v.at[g],
                out_hbm.at[pl.ds(base + g * GATHER_CHUNK, GATHER_CHUNK)], wsem))
        for w in writes:
            w.wait()

    return gather_k(table, idx)


def kernel(distances, table):
    idx = _centrality_indices(distances)
    out = _sc_gather(table, idx)
    return out.reshape(B, N, D_MODEL)
